# baseline (speedup 1.0000x reference)
import jax
import jax.numpy as jnp
from jax import lax
from jax.experimental import pallas as pl
from jax.experimental.pallas import tpu as pltpu
from jax.experimental.pallas import tpu_sc as plsc


_CH = 16
_NBX = 3


def kernel(x, pos_embedding):
    batch, seq_len, d = x.shape
    rows = batch * seq_len
    x2 = x.reshape(rows, d)

    mesh = plsc.VectorSubcoreMesh(core_axis_name="c", subcore_axis_name="s")
    nw = mesh.num_cores * mesh.num_subcores
    nc = mesh.num_cores
    s_per_w = seq_len // nw
    n_chunks = s_per_w // _CH
    dg = d // 16
    groups = _CH * dg

    def body(x_hbm, pe_hbm, o_hbm, bpe_refs, bx_refs, pe_sems, x_sems, o_sems):
        wid = lax.axis_index("s") * nc + lax.axis_index("c")
        s0 = wid * s_per_w

        def pe_load(ci):
            return pltpu.async_copy(
                pe_hbm.at[pl.ds(s0 + ci * _CH, _CH)],
                bpe_refs[ci % 2], pe_sems[ci % 2])

        def x_load(t):
            ci, b = t // batch, t % batch
            off = b * seq_len + s0 + ci * _CH
            return pltpu.async_copy(
                x_hbm.at[pl.ds(off, _CH)], bx_refs[t % _NBX], x_sems[t % _NBX])

        def o_store(t):
            ci, b = t // batch, t % batch
            off = b * seq_len + s0 + ci * _CH
            return pltpu.async_copy(
                bx_refs[t % _NBX], o_hbm.at[pl.ds(off, _CH)], o_sems[t % _NBX])

        n_steps = n_chunks * batch
        pe_cps = [pe_load(0), pe_load(1)]
        x_cps = [x_load(t) for t in range(_NBX)]
        o_cps = [None] * _NBX

        for t in range(n_steps):
            ci = t // batch
            k = t % _NBX
            if t % batch == 0:
                pe_cps[ci % 2].wait()
            x_cps[k].wait()

            bx = bx_refs[k]
            bpe = bpe_refs[ci % 2]

            @plsc.parallel_loop(0, groups, unroll=16)
            def _(g, _bx=bx, _bpe=bpe):
                r = g // dg
                c = lax.rem(g, dg) * 16
                pev = _bpe[r, pl.ds(c, 16)]
                plsc.addupdate(_bx.at[r, pl.ds(c, 16)], pev)

            o_cps[k] = o_store(t)

            tn = t + _NBX
            if tn < n_steps:
                if o_cps[tn % _NBX] is not None:
                    o_cps[tn % _NBX].wait()
                x_cps[tn % _NBX] = x_load(tn)
            if t % batch == batch - 1 and ci + 2 < n_chunks:
                pe_cps[ci % 2] = pe_load(ci + 2)

        for k in range(_NBX):
            if o_cps[k] is not None:
                o_cps[k].wait()

    sc_add = pl.kernel(
        body,
        out_type=jax.ShapeDtypeStruct((rows, d), x.dtype),
        mesh=mesh,
        scratch_types=[
            [pltpu.VMEM((_CH, d), jnp.float32) for _ in range(2)],
            [pltpu.VMEM((_CH, d), jnp.float32) for _ in range(_NBX)],
            [pltpu.SemaphoreType.DMA for _ in range(2)],
            [pltpu.SemaphoreType.DMA for _ in range(_NBX)],
            [pltpu.SemaphoreType.DMA for _ in range(_NBX)],
        ],
        compiler_params=pltpu.CompilerParams(
            use_tc_tiling_on_sc=True, has_side_effects=True
        ),
    )
    _ = sc_add(x2, pos_embedding)

    def _tc_body(x_ref, pe_ref, o_ref):
        o_ref[...] = x_ref[...] + pe_ref[...]

    bs = 2048
    tc_out = pl.pallas_call(
        _tc_body,
        grid=(seq_len // bs, batch),
        in_specs=[
            pl.BlockSpec((1, bs, d), lambda s, b: (b, s, 0)),
            pl.BlockSpec((bs, d), lambda s, b: (s, 0)),
        ],
        out_specs=pl.BlockSpec((1, bs, d), lambda s, b: (b, s, 0)),
        out_shape=jax.ShapeDtypeStruct((batch, seq_len, d), x.dtype),
    )(x, pos_embedding)
    return tc_out


# speedup vs baseline: 1.0615x; 1.0615x over previous
import jax
import jax.numpy as jnp
from jax import lax
from jax.experimental import pallas as pl
from jax.experimental.pallas import tpu as pltpu
from jax.experimental.pallas import tpu_sc as plsc


_CH = 16
_NBX = 3


def _sc_one_batch(x2, pos_embedding, row_base, seq_len, d):
    mesh = plsc.VectorSubcoreMesh(core_axis_name="c", subcore_axis_name="s")
    nw = mesh.num_cores * mesh.num_subcores
    nc = mesh.num_cores
    s_per_w = seq_len // nw
    n_chunks = s_per_w // _CH
    dg = d // 16
    groups = _CH * dg

    def body(x_hbm, pe_hbm, o_hbm, bpe_refs, bx_refs, pe_sems, x_sems, o_sems):
        wid = lax.axis_index("s") * nc + lax.axis_index("c")
        s0 = wid * s_per_w

        def pe_load(ci):
            return pltpu.async_copy(
                pe_hbm.at[pl.ds(s0 + ci * _CH, _CH)],
                bpe_refs[ci % 2], pe_sems[ci % 2])

        def x_load(t):
            off = row_base + s0 + t * _CH
            return pltpu.async_copy(
                x_hbm.at[pl.ds(off, _CH)], bx_refs[t % _NBX], x_sems[t % _NBX])

        def o_store(t):
            off = s0 + t * _CH
            return pltpu.async_copy(
                bx_refs[t % _NBX], o_hbm.at[pl.ds(off, _CH)], o_sems[t % _NBX])

        pe_cps = [pe_load(0), pe_load(1)]
        x_cps = [x_load(t) for t in range(_NBX)]
        o_cps = [None] * _NBX

        for t in range(n_chunks):
            k = t % _NBX
            pe_cps[t % 2].wait()
            x_cps[k].wait()

            bx = bx_refs[k]
            bpe = bpe_refs[t % 2]

            @plsc.parallel_loop(0, groups, unroll=16)
            def _(g, _bx=bx, _bpe=bpe):
                r = g // dg
                c = lax.rem(g, dg) * 16
                pev = _bpe[r, pl.ds(c, 16)]
                plsc.addupdate(_bx.at[r, pl.ds(c, 16)], pev)

            o_cps[k] = o_store(t)

            tn = t + _NBX
            if tn < n_chunks:
                if o_cps[tn % _NBX] is not None:
                    o_cps[tn % _NBX].wait()
                x_cps[tn % _NBX] = x_load(tn)
            if t + 2 < n_chunks:
                pe_cps[t % 2] = pe_load(t + 2)

        for k in range(_NBX):
            if o_cps[k] is not None:
                o_cps[k].wait()

    sc_add = pl.kernel(
        body,
        out_type=jax.ShapeDtypeStruct((seq_len, d), x2.dtype),
        mesh=mesh,
        scratch_types=[
            [pltpu.VMEM((_CH, d), jnp.float32) for _ in range(2)],
            [pltpu.VMEM((_CH, d), jnp.float32) for _ in range(_NBX)],
            [pltpu.SemaphoreType.DMA for _ in range(2)],
            [pltpu.SemaphoreType.DMA for _ in range(_NBX)],
            [pltpu.SemaphoreType.DMA for _ in range(_NBX)],
        ],
        compiler_params=pltpu.CompilerParams(use_tc_tiling_on_sc=True),
    )
    return sc_add(x2, pos_embedding)


def _tc_body(x_ref, pe_ref, o_ref):
    o_ref[...] = x_ref[...] + pe_ref[...]


def kernel(x, pos_embedding):
    batch, seq_len, d = x.shape
    x2 = x.reshape(batch * seq_len, d)

    sc_out = _sc_one_batch(x2, pos_embedding, (batch - 1) * seq_len, seq_len, d)

    bs = 2048
    tc_out = pl.pallas_call(
        _tc_body,
        grid=(seq_len // bs, batch - 1),
        in_specs=[
            pl.BlockSpec((1, bs, d), lambda s, b: (b, s, 0)),
            pl.BlockSpec((bs, d), lambda s, b: (s, 0)),
        ],
        out_specs=pl.BlockSpec((1, bs, d), lambda s, b: (b, s, 0)),
        out_shape=jax.ShapeDtypeStruct((batch - 1, seq_len, d), x.dtype),
    )(x, pos_embedding)

    return jnp.concatenate([tc_out, sc_out[None]], axis=0)


# speedup vs baseline: 1.3134x; 1.2374x over previous
import jax
import jax.numpy as jnp
from jax import lax
from jax.experimental import pallas as pl
from jax.experimental.pallas import tpu as pltpu
from jax.experimental.pallas import tpu_sc as plsc


_CH = 16
_NBX = 3


def kernel(x, pos_embedding):
    batch, seq_len, d = x.shape
    rows = batch * seq_len
    x2 = x.reshape(rows, d)

    mesh = plsc.VectorSubcoreMesh(core_axis_name="c", subcore_axis_name="s")
    nw = mesh.num_cores * mesh.num_subcores
    nc = mesh.num_cores
    s_per_w = seq_len // nw
    n_chunks = s_per_w // _CH
    dg = d // 16
    groups = _CH * dg

    def body(x_hbm, pe_hbm, o_hbm, bpe_refs, bx_refs, pe_sems, x_sems, o_sems):
        wid = lax.axis_index("s") * nc + lax.axis_index("c")
        s0 = wid * s_per_w

        def pe_load(ci):
            return pltpu.async_copy(
                pe_hbm.at[pl.ds(s0 + ci * _CH, _CH)],
                bpe_refs[ci % 2], pe_sems[ci % 2])

        def x_load(t):
            ci, b = t // batch, t % batch
            off = b * seq_len + s0 + ci * _CH
            return pltpu.async_copy(
                x_hbm.at[pl.ds(off, _CH)], bx_refs[t % _NBX], x_sems[t % _NBX])

        def o_store(t):
            ci, b = t // batch, t % batch
            off = b * seq_len + s0 + ci * _CH
            return pltpu.async_copy(
                bx_refs[t % _NBX], o_hbm.at[pl.ds(off, _CH)], o_sems[t % _NBX])

        n_steps = n_chunks * batch
        pe_cps = [pe_load(0), pe_load(1)]
        x_cps = [x_load(t) for t in range(_NBX)]
        o_cps = [None] * _NBX

        for t in range(n_steps):
            ci = t // batch
            k = t % _NBX
            if t % batch == 0:
                pe_cps[ci % 2].wait()
            x_cps[k].wait()

            bx = bx_refs[k]
            bpe = bpe_refs[ci % 2]

            @plsc.parallel_loop(0, groups, unroll=32)
            def _(g, _bx=bx, _bpe=bpe):
                r = g // dg
                c = lax.rem(g, dg) * 16
                pev = _bpe[r, pl.ds(c, 16)]
                plsc.addupdate(_bx.at[r, pl.ds(c, 16)], pev)

            o_cps[k] = o_store(t)

            tn = t + _NBX
            if tn < n_steps:
                if o_cps[tn % _NBX] is not None:
                    o_cps[tn % _NBX].wait()
                x_cps[tn % _NBX] = x_load(tn)
            if t % batch == batch - 1 and ci + 2 < n_chunks:
                pe_cps[ci % 2] = pe_load(ci + 2)

        for k in range(_NBX):
            if o_cps[k] is not None:
                o_cps[k].wait()

    sc_add = pl.kernel(
        body,
        out_type=jax.ShapeDtypeStruct((rows, d), x.dtype),
        mesh=mesh,
        scratch_types=[
            [pltpu.VMEM((_CH, d), jnp.float32) for _ in range(2)],
            [pltpu.VMEM((_CH, d), jnp.float32) for _ in range(_NBX)],
            [pltpu.SemaphoreType.DMA for _ in range(2)],
            [pltpu.SemaphoreType.DMA for _ in range(_NBX)],
            [pltpu.SemaphoreType.DMA for _ in range(_NBX)],
        ],
        compiler_params=pltpu.CompilerParams(use_tc_tiling_on_sc=True),
    )
    out = sc_add(x2, pos_embedding)
    return out.reshape(batch, seq_len, d)


# speedup vs baseline: 2.3529x; 1.7914x over previous
import jax
import jax.numpy as jnp
from jax.experimental import pallas as pl


_BS = 2048
_DS = 512


def _add_kernel(x_ref, pe_ref, o_ref):
    o_ref[...] = x_ref[...] + pe_ref[...]


def kernel(x, pos_embedding):
    batch, seq_len, d = x.shape
    return pl.pallas_call(
        _add_kernel,
        grid=(d // _DS, seq_len // _BS, batch),
        in_specs=[
            pl.BlockSpec((1, _BS, _DS), lambda j, s, b: (b, s, j)),
            pl.BlockSpec((_BS, _DS), lambda j, s, b: (s, j)),
        ],
        out_specs=pl.BlockSpec((1, _BS, _DS), lambda j, s, b: (b, s, j)),
        out_shape=jax.ShapeDtypeStruct((batch, seq_len, d), x.dtype),
    )(x, pos_embedding)


# speedup vs baseline: 2.5216x; 1.0717x over previous
import jax
import jax.numpy as jnp
from jax.experimental import pallas as pl


_BS = 2048


def _add_kernel(x_ref, pe_ref, o_ref):
    o_ref[...] = x_ref[...] + pe_ref[...]


def kernel(x, pos_embedding):
    batch, seq_len, d = x.shape
    return pl.pallas_call(
        _add_kernel,
        grid=(seq_len // _BS, batch),
        in_specs=[
            pl.BlockSpec((1, _BS, d), lambda s, b: (b, s, 0)),
            pl.BlockSpec((_BS, d), lambda s, b: (s, 0)),
        ],
        out_specs=pl.BlockSpec((1, _BS, d), lambda s, b: (b, s, 0)),
        out_shape=jax.ShapeDtypeStruct((batch, seq_len, d), x.dtype),
    )(x, pos_embedding)
